# baseline (speedup 1.0000x reference)
import functools

import jax
import jax.numpy as jnp
from jax import lax
from jax.experimental import pallas as pl
from jax.experimental.pallas import tpu as pltpu
from jax.experimental.pallas import tpu_sc as plsc

N = 10000
E = 320000
D = 128
M = 64
EPS = 1e-5

BLK = 1000
GRID = N // BLK

NC, NS = 2, 16
NW = NC * NS
NPAD = 10112
RPT = NPAD // NS
CH = 128
CHUNKS = 160
SLABS = CHUNKS + 4
EPT = CHUNKS * CH
P = NW * EPT
IB = 32


def _mm(x, w):
    return lax.dot_general(x, w, (((1,), (1,)), ((), ())),
                           preferred_element_type=jnp.float32)


def _scatter_body(xn_hbm, src_hbm, dst_hbm, zeros_hbm, out_hbm,
                  s0, s1, s2, s3, d0, d1, d2, d3, rows0, rows1, acc_sh,
                  gi0, gi1, gi2, gi3, gg0, gg1, ss0, ss1):
    c = lax.axis_index("c")
    s = lax.axis_index("s")
    wid = s * NC + c
    r0 = s * RPT
    base = wid * SLABS * CH
    sbuf = (s0, s1, s2, s3)
    dbuf = (d0, d1, d2, d3)
    isem = (gi0, gi1, gi2, gi3)

    def load_idx(k, slab):
        off = base + slab * CH
        pltpu.async_copy(src_hbm.at[pl.ds(off, CH)], sbuf[k], isem[k])
        pltpu.async_copy(dst_hbm.at[pl.ds(off, CH)], dbuf[k], isem[k])

    def wait_idx(k):
        pltpu.make_async_copy(src_hbm.at[pl.ds(base, CH)], sbuf[k], isem[k]).wait()
        pltpu.make_async_copy(dst_hbm.at[pl.ds(base, CH)], dbuf[k], isem[k]).wait()

    def start_gather(k, rbuf, gsem):
        pltpu.async_copy(xn_hbm.at[sbuf[k]], rbuf, gsem)

    def wait_rows(rbuf, sem):
        pltpu.make_async_copy(zeros_hbm.at[pl.ds(0, CH)], rbuf, sem).wait()

    def start_scatter(k, rbuf, ssem):
        pltpu.async_copy(rbuf, acc_sh.at[dbuf[k]], ssem, add=True)

    for k in range(4):
        load_idx(k, k)
    pltpu.sync_copy(zeros_hbm.at[pl.ds(r0, RPT)], acc_sh.at[pl.ds(r0, RPT)])
    plsc.subcore_barrier()
    wait_idx(0)
    start_gather(0, rows0, gg0)

    def chunk_step(kk, kn, kp, rb, rbn, gs, gsn, ss, ssn, slab_next,
                   first=False):
        wait_rows(rb, gs)
        wait_idx(kn)
        if not first:
            wait_rows(rbn, ssn)
            load_idx(kp, slab_next)
        start_gather(kn, rbn, gsn)
        start_scatter(kk, rb, ss)

    def step(j, carry):
        c0 = 4 * j
        chunk_step(0, 1, 3, rows0, rows1, gg0, gg1, ss0, ss1, c0 + 3)
        chunk_step(1, 2, 0, rows1, rows0, gg1, gg0, ss1, ss0, c0 + 4)
        chunk_step(2, 3, 1, rows0, rows1, gg0, gg1, ss0, ss1, c0 + 5)
        chunk_step(3, 0, 2, rows1, rows0, gg1, gg0, ss1, ss0, c0 + 6)
        return carry

    chunk_step(0, 1, 3, rows0, rows1, gg0, gg1, ss0, ss1, 3, first=True)
    chunk_step(1, 2, 0, rows1, rows0, gg1, gg0, ss1, ss0, 4)
    chunk_step(2, 3, 1, rows0, rows1, gg0, gg1, ss0, ss1, 5)
    chunk_step(3, 0, 2, rows1, rows0, gg1, gg0, ss1, ss0, 6)
    lax.fori_loop(1, CHUNKS // 4, step, 0)
    wait_rows(rows0, gg0)
    wait_rows(rows1, ss1)
    wait_idx(1)
    wait_idx(2)
    plsc.subcore_barrier()
    pltpu.sync_copy(acc_sh.at[pl.ds(r0, RPT)], out_hbm.at[c, pl.ds(r0, RPT)])


_edge_scatter = pl.kernel(
    _scatter_body,
    out_type=jax.ShapeDtypeStruct((NC, NPAD, D), jnp.float32),
    mesh=plsc.VectorSubcoreMesh(core_axis_name="c", subcore_axis_name="s"),
    scratch_types=[
        pltpu.VMEM((CH,), jnp.int32),
        pltpu.VMEM((CH,), jnp.int32),
        pltpu.VMEM((CH,), jnp.int32),
        pltpu.VMEM((CH,), jnp.int32),
        pltpu.VMEM((CH,), jnp.int32),
        pltpu.VMEM((CH,), jnp.int32),
        pltpu.VMEM((CH,), jnp.int32),
        pltpu.VMEM((CH,), jnp.int32),
        pltpu.VMEM((CH, D), jnp.float32),
        pltpu.VMEM((CH, D), jnp.float32),
        pltpu.VMEM_SHARED((NPAD, D), jnp.float32),
        pltpu.SemaphoreType.DMA,
        pltpu.SemaphoreType.DMA,
        pltpu.SemaphoreType.DMA,
        pltpu.SemaphoreType.DMA,
        pltpu.SemaphoreType.DMA,
        pltpu.SemaphoreType.DMA,
        pltpu.SemaphoreType.DMA,
        pltpu.SemaphoreType.DMA,
    ],
)


def _dense2_body(x_ref, w0_ref, b0_ref, w1_ref, b1_ref, out0_ref, xn_ref):
    x = x_ref[...]
    out0_ref[...] = _mm(x, w0_ref[...]) + b0_ref[...]
    xn_ref[...] = _mm(x, w1_ref[...]) + b1_ref[...]


def _dense2(x, w0, b0, w1, b1):
    full = lambda shape: pl.BlockSpec(shape, lambda i: (0, 0))
    return pl.pallas_call(
        _dense2_body,
        grid=(GRID,),
        in_specs=[
            pl.BlockSpec((BLK, D), lambda i: (i, 0)),
            full((D, D)), full((1, D)), full((D, D)), full((1, D)),
        ],
        out_specs=[
            pl.BlockSpec((BLK, D), lambda i: (i, 0)),
            pl.BlockSpec((BLK, D), lambda i: (i, 0)),
        ],
        out_shape=[
            jax.ShapeDtypeStruct((N, D), jnp.float32),
            jax.ShapeDtypeStruct((N, D), jnp.float32),
        ],
    )(x, w0, b0, w1, b1)


def _combine_body(out0_ref, acc0_ref, acc1_ref, h_ref, stats_ref, sacc):
    i = pl.program_id(0)
    h = out0_ref[...] + acc0_ref[0] + acc1_ref[0]
    h_ref[...] = h

    @pl.when(i == 0)
    def _():
        sacc[...] = jnp.zeros_like(sacc)

    s1 = jnp.sum(h, axis=0, keepdims=True)
    s2 = jnp.sum(h * h, axis=0, keepdims=True)
    sacc[...] += jnp.concatenate([s1, s2], axis=0)

    @pl.when(i == GRID - 1)
    def _():
        stats_ref[...] = sacc[...]


def _combine(out0, acc):
    return pl.pallas_call(
        _combine_body,
        grid=(GRID,),
        in_specs=[
            pl.BlockSpec((BLK, D), lambda i: (i, 0)),
            pl.BlockSpec((1, BLK, D), lambda i: (0, i, 0)),
            pl.BlockSpec((1, BLK, D), lambda i: (1, i, 0)),
        ],
        out_specs=[
            pl.BlockSpec((BLK, D), lambda i: (i, 0)),
            pl.BlockSpec((2, D), lambda i: (0, 0)),
        ],
        out_shape=[
            jax.ShapeDtypeStruct((N, D), jnp.float32),
            jax.ShapeDtypeStruct((2, D), jnp.float32),
        ],
        scratch_shapes=[pltpu.VMEM((2, D), jnp.float32)],
    )(out0, acc, acc)


def _bn_apply(h, stats, g, b):
    mu = stats[0:1] / N
    var = stats[1:2] / N - mu * mu
    inv = lax.rsqrt(var + EPS)
    return jnp.maximum(g * (h - mu) * inv + b, 0.0)


def _bnrelu_dense2_body(h_ref, stats_ref, g_ref, b_ref,
                        w0_ref, b0_ref, w1_ref, b1_ref, out0_ref, xn_ref):
    x = _bn_apply(h_ref[...], stats_ref[...], g_ref[...], b_ref[...])
    out0_ref[...] = _mm(x, w0_ref[...]) + b0_ref[...]
    xn_ref[...] = _mm(x, w1_ref[...]) + b1_ref[...]


def _bnrelu_dense2(h, stats, g, b, w0, b0, w1, b1):
    full = lambda shape: pl.BlockSpec(shape, lambda i: (0, 0))
    return pl.pallas_call(
        _bnrelu_dense2_body,
        grid=(GRID,),
        in_specs=[
            pl.BlockSpec((BLK, D), lambda i: (i, 0)),
            full((2, D)), full((1, D)), full((1, D)),
            full((D, D)), full((1, D)), full((D, D)), full((1, D)),
        ],
        out_specs=[
            pl.BlockSpec((BLK, D), lambda i: (i, 0)),
            pl.BlockSpec((BLK, D), lambda i: (i, 0)),
        ],
        out_shape=[
            jax.ShapeDtypeStruct((N, D), jnp.float32),
            jax.ShapeDtypeStruct((N, D), jnp.float32),
        ],
    )(h, stats, g, b, w0, b0, w1, b1)


def _final_body(h_ref, stats_ref, g_ref, b_ref, idx_ref,
                fc1_w_ref, fc1_b_ref,
                st1_w_ref, st1_b_ref, se1_w_ref, se1_b_ref,
                fu1_w_ref, fu1_b_ref, ae1_w_ref, ae1_b_ref,
                st2_w_ref, st2_b_ref, se2_w_ref, se2_b_ref,
                fu2_w_ref, fu2_b_ref, ae2_w_ref, ae2_b_ref,
                st_ref, se_ref, fu_ref, ae_ref, seg_acc, cnt_acc):
    i = pl.program_id(0)
    x = _bn_apply(h_ref[...], stats_ref[...], g_ref[...], b_ref[...])

    @pl.when(i == 0)
    def _():
        seg_acc[...] = jnp.zeros_like(seg_acc)
        cnt_acc[...] = jnp.zeros_like(cnt_acc)

    idx = idx_ref[0]
    seg_ids = lax.broadcasted_iota(jnp.int32, (M, BLK), 0)
    onehot = (seg_ids == idx).astype(jnp.float32)
    seg_acc[...] += jnp.dot(onehot, x, preferred_element_type=jnp.float32)
    cnt = jnp.sum(onehot, axis=1, keepdims=True)
    cnt_acc[...] += jnp.broadcast_to(cnt, (M, D))

    @pl.when(i == GRID - 1)
    def _():
        mean = seg_acc[...] / jnp.maximum(cnt_acc[...], 1.0)
        relu = lambda v: jnp.maximum(v, 0.0)
        o = relu(_mm(mean, fc1_w_ref[...]) + fc1_b_ref[...])
        st_ref[...] = _mm(relu(_mm(o, st1_w_ref[...]) + st1_b_ref[...]),
                          st2_w_ref[...]) + st2_b_ref[...]
        se_ref[...] = _mm(relu(_mm(o, se1_w_ref[...]) + se1_b_ref[...]),
                          se2_w_ref[...]) + se2_b_ref[...]
        fu_ref[...] = _mm(relu(_mm(o, fu1_w_ref[...]) + fu1_b_ref[...]),
                          fu2_w_ref[...]) + fu2_b_ref[...]
        ae_ref[...] = _mm(relu(_mm(o, ae1_w_ref[...]) + ae1_b_ref[...]),
                          ae2_w_ref[...]) + ae2_b_ref[...]


def _final(h, stats, g, b, idx3, fc1_w, fc1_b,
           st1_w, st1_b, se1_w, se1_b, fu1_w, fu1_b, ae1_w, ae1_b,
           st2_w, st2_b, se2_w, se2_b, fu2_w, fu2_b, ae2_w, ae2_b):
    full = lambda shape: pl.BlockSpec(shape, lambda i: tuple(0 for _ in shape))
    return pl.pallas_call(
        _final_body,
        grid=(GRID,),
        in_specs=[
            pl.BlockSpec((BLK, D), lambda i: (i, 0)),
            full((2, D)), full((1, D)), full((1, D)),
            pl.BlockSpec((1, 1, BLK), lambda i: (i, 0, 0)),
            full((512, D)), full((1, 512)),
            full((256, 512)), full((1, 256)), full((256, 512)), full((1, 256)),
            full((256, 512)), full((1, 256)), full((256, 512)), full((1, 256)),
            full((3, 256)), full((1, 3)), full((2, 256)), full((1, 2)),
            full((4, 256)), full((1, 4)), full((5, 256)), full((1, 5)),
        ],
        out_specs=[
            pl.BlockSpec((M, 3), lambda i: (0, 0)),
            pl.BlockSpec((M, 2), lambda i: (0, 0)),
            pl.BlockSpec((M, 4), lambda i: (0, 0)),
            pl.BlockSpec((M, 5), lambda i: (0, 0)),
        ],
        out_shape=[
            jax.ShapeDtypeStruct((M, 3), jnp.float32),
            jax.ShapeDtypeStruct((M, 2), jnp.float32),
            jax.ShapeDtypeStruct((M, 4), jnp.float32),
            jax.ShapeDtypeStruct((M, 5), jnp.float32),
        ],
        scratch_shapes=[pltpu.VMEM((M, D), jnp.float32),
                        pltpu.VMEM((M, D), jnp.float32)],
    )(h, stats, g, b, idx3, fc1_w, fc1_b,
      st1_w, st1_b, se1_w, se1_b, fu1_w, fu1_b, ae1_w, ae1_b,
      st2_w, st2_b, se2_w, se2_b, fu2_w, fu2_b, ae2_w, ae2_b)


def kernel(verts, edges, verts_idx, g0_w0, g0_b0, g0_w1, g0_b1,
           g1_w0, g1_b0, g1_w1, g1_b1, bn0_g, bn0_b, bn1_g, bn1_b,
           fc1_w, fc1_b, st1_w, st1_b, se1_w, se1_b, fu1_w, fu1_b,
           ae1_w, ae1_b, st2_w, st2_b, se2_w, se2_b, fu2_w, fu2_b,
           ae2_w, ae2_b):
    r = lambda v: v.reshape(1, -1)
    e0 = edges[:, 0]
    e1 = edges[:, 1]
    pad = P - 2 * E
    src = jnp.concatenate([e1, e0, jnp.zeros((pad,), jnp.int32)])
    dst = jnp.concatenate([e0, e1, N + (jnp.arange(pad) % (NPAD - N))])
    dst = dst.astype(jnp.int32)
    cpad = jnp.zeros((NW, SLABS - CHUNKS, CH), jnp.int32)
    srcp = jnp.concatenate([src.reshape(NW, CHUNKS, CH), cpad],
                           axis=1).reshape(-1)
    dstp = jnp.concatenate([dst.reshape(NW, CHUNKS, CH), cpad],
                           axis=1).reshape(-1)
    zeros = jnp.zeros((NPAD, D), jnp.float32)
    idx3 = verts_idx.reshape(GRID, 1, BLK)

    out0, xn0 = _dense2(verts, g0_w0, r(g0_b0), g0_w1, r(g0_b1))
    acc = _edge_scatter(xn0, srcp, dstp, zeros)
    h1, stats1 = _combine(out0, acc)
    out1, xn1 = _bnrelu_dense2(h1, stats1, r(bn0_g), r(bn0_b),
                               g1_w0, r(g1_b0), g1_w1, r(g1_b1))
    acc2 = _edge_scatter(xn1, srcp, dstp, zeros)
    h2, stats2 = _combine(out1, acc2)
    return _final(h2, stats2, r(bn1_g), r(bn1_b), idx3, fc1_w, r(fc1_b),
                  st1_w, r(st1_b), se1_w, r(se1_b), fu1_w, r(fu1_b),
                  ae1_w, r(ae1_b), st2_w, r(st2_b), se2_w, r(se2_b),
                  fu2_w, r(fu2_b), ae2_w, r(ae2_b))


# speedup vs baseline: 2.2190x; 2.2190x over previous
import functools

import jax
import jax.numpy as jnp
from jax import lax
from jax.experimental import pallas as pl
from jax.experimental.pallas import tpu as pltpu
from jax.experimental.pallas import tpu_sc as plsc

N = 10000
E = 320000
D = 128
M = 64
EPS = 1e-5

BLK = 1000
GRID = N // BLK

NC, NS = 2, 16
NW = NC * NS
NPAD = 10112
RPT = NPAD // NS
CH = 128
CHUNKS = 157
EPT = CHUNKS * CH
P = NW * EPT
IB = 32


def _mm(x, w):
    return lax.dot_general(x, w, (((1,), (1,)), ((), ())),
                           preferred_element_type=jnp.float32)


def _scatter_body(xn_hbm, src_hbm, dst_hbm, zeros_hbm, out_hbm,
                  src_v, dst_v, rows_v, acc_sh, gi, gg):
    c = lax.axis_index("c")
    s = lax.axis_index("s")
    wid = s * NC + c
    r0 = s * RPT
    pltpu.sync_copy(zeros_hbm.at[pl.ds(r0, RPT)], acc_sh.at[pl.ds(r0, RPT)])
    plsc.subcore_barrier()
    base = wid * EPT

    def chunk(i, carry):
        boff = base + i * CH
        a = pltpu.async_copy(src_hbm.at[pl.ds(boff, CH)], src_v, gi)
        b = pltpu.async_copy(dst_hbm.at[pl.ds(boff, CH)], dst_v, gi)
        a.wait()
        b.wait()
        pltpu.async_copy(xn_hbm.at[src_v], rows_v, gg).wait()
        pltpu.sync_copy(rows_v, acc_sh.at[dst_v], add=True)
        return carry

    lax.fori_loop(0, CHUNKS, chunk, 0)
    plsc.subcore_barrier()
    pltpu.sync_copy(acc_sh.at[pl.ds(r0, RPT)], out_hbm.at[c, pl.ds(r0, RPT)])


_edge_scatter = pl.kernel(
    _scatter_body,
    out_type=jax.ShapeDtypeStruct((NC, NPAD, D), jnp.float32),
    mesh=plsc.VectorSubcoreMesh(core_axis_name="c", subcore_axis_name="s"),
    scratch_types=[
        pltpu.VMEM((CH,), jnp.int32),
        pltpu.VMEM((CH,), jnp.int32),
        pltpu.VMEM((CH, D), jnp.float32),
        pltpu.VMEM_SHARED((NPAD, D), jnp.float32),
        pltpu.SemaphoreType.DMA,
        pltpu.SemaphoreType.DMA,
    ],
)


def _dense2_body(x_ref, w0_ref, b0_ref, w1_ref, b1_ref, out0_ref, xn_ref):
    x = x_ref[...]
    out0_ref[...] = _mm(x, w0_ref[...]) + b0_ref[...]
    xn_ref[...] = _mm(x, w1_ref[...]) + b1_ref[...]


def _dense2(x, w0, b0, w1, b1):
    full = lambda shape: pl.BlockSpec(shape, lambda i: (0, 0))
    return pl.pallas_call(
        _dense2_body,
        grid=(GRID,),
        in_specs=[
            pl.BlockSpec((BLK, D), lambda i: (i, 0)),
            full((D, D)), full((1, D)), full((D, D)), full((1, D)),
        ],
        out_specs=[
            pl.BlockSpec((BLK, D), lambda i: (i, 0)),
            pl.BlockSpec((BLK, D), lambda i: (i, 0)),
        ],
        out_shape=[
            jax.ShapeDtypeStruct((N, D), jnp.float32),
            jax.ShapeDtypeStruct((N, D), jnp.float32),
        ],
    )(x, w0, b0, w1, b1)


def _combine_body(out0_ref, acc0_ref, acc1_ref, h_ref, stats_ref, sacc):
    i = pl.program_id(0)
    h = out0_ref[...] + acc0_ref[0] + acc1_ref[0]
    h_ref[...] = h

    @pl.when(i == 0)
    def _():
        sacc[...] = jnp.zeros_like(sacc)

    s1 = jnp.sum(h, axis=0, keepdims=True)
    s2 = jnp.sum(h * h, axis=0, keepdims=True)
    sacc[...] += jnp.concatenate([s1, s2], axis=0)

    @pl.when(i == GRID - 1)
    def _():
        stats_ref[...] = sacc[...]


def _combine(out0, acc):
    return pl.pallas_call(
        _combine_body,
        grid=(GRID,),
        in_specs=[
            pl.BlockSpec((BLK, D), lambda i: (i, 0)),
            pl.BlockSpec((1, BLK, D), lambda i: (0, i, 0)),
            pl.BlockSpec((1, BLK, D), lambda i: (1, i, 0)),
        ],
        out_specs=[
            pl.BlockSpec((BLK, D), lambda i: (i, 0)),
            pl.BlockSpec((2, D), lambda i: (0, 0)),
        ],
        out_shape=[
            jax.ShapeDtypeStruct((N, D), jnp.float32),
            jax.ShapeDtypeStruct((2, D), jnp.float32),
        ],
        scratch_shapes=[pltpu.VMEM((2, D), jnp.float32)],
    )(out0, acc, acc)


def _bn_apply(h, stats, g, b):
    mu = stats[0:1] / N
    var = stats[1:2] / N - mu * mu
    inv = lax.rsqrt(var + EPS)
    return jnp.maximum(g * (h - mu) * inv + b, 0.0)


def _bnrelu_dense2_body(h_ref, stats_ref, g_ref, b_ref,
                        w0_ref, b0_ref, w1_ref, b1_ref, out0_ref, xn_ref):
    x = _bn_apply(h_ref[...], stats_ref[...], g_ref[...], b_ref[...])
    out0_ref[...] = _mm(x, w0_ref[...]) + b0_ref[...]
    xn_ref[...] = _mm(x, w1_ref[...]) + b1_ref[...]


def _bnrelu_dense2(h, stats, g, b, w0, b0, w1, b1):
    full = lambda shape: pl.BlockSpec(shape, lambda i: (0, 0))
    return pl.pallas_call(
        _bnrelu_dense2_body,
        grid=(GRID,),
        in_specs=[
            pl.BlockSpec((BLK, D), lambda i: (i, 0)),
            full((2, D)), full((1, D)), full((1, D)),
            full((D, D)), full((1, D)), full((D, D)), full((1, D)),
        ],
        out_specs=[
            pl.BlockSpec((BLK, D), lambda i: (i, 0)),
            pl.BlockSpec((BLK, D), lambda i: (i, 0)),
        ],
        out_shape=[
            jax.ShapeDtypeStruct((N, D), jnp.float32),
            jax.ShapeDtypeStruct((N, D), jnp.float32),
        ],
    )(h, stats, g, b, w0, b0, w1, b1)


def _final_body(h_ref, stats_ref, g_ref, b_ref, idx_ref,
                fc1_w_ref, fc1_b_ref,
                st1_w_ref, st1_b_ref, se1_w_ref, se1_b_ref,
                fu1_w_ref, fu1_b_ref, ae1_w_ref, ae1_b_ref,
                st2_w_ref, st2_b_ref, se2_w_ref, se2_b_ref,
                fu2_w_ref, fu2_b_ref, ae2_w_ref, ae2_b_ref,
                st_ref, se_ref, fu_ref, ae_ref, seg_acc, cnt_acc):
    i = pl.program_id(0)
    x = _bn_apply(h_ref[...], stats_ref[...], g_ref[...], b_ref[...])

    @pl.when(i == 0)
    def _():
        seg_acc[...] = jnp.zeros_like(seg_acc)
        cnt_acc[...] = jnp.zeros_like(cnt_acc)

    idx = idx_ref[0]
    seg_ids = lax.broadcasted_iota(jnp.int32, (M, BLK), 0)
    onehot = (seg_ids == idx).astype(jnp.float32)
    seg_acc[...] += jnp.dot(onehot, x, preferred_element_type=jnp.float32)
    cnt = jnp.sum(onehot, axis=1, keepdims=True)
    cnt_acc[...] += jnp.broadcast_to(cnt, (M, D))

    @pl.when(i == GRID - 1)
    def _():
        mean = seg_acc[...] / jnp.maximum(cnt_acc[...], 1.0)
        relu = lambda v: jnp.maximum(v, 0.0)
        o = relu(_mm(mean, fc1_w_ref[...]) + fc1_b_ref[...])
        st_ref[...] = _mm(relu(_mm(o, st1_w_ref[...]) + st1_b_ref[...]),
                          st2_w_ref[...]) + st2_b_ref[...]
        se_ref[...] = _mm(relu(_mm(o, se1_w_ref[...]) + se1_b_ref[...]),
                          se2_w_ref[...]) + se2_b_ref[...]
        fu_ref[...] = _mm(relu(_mm(o, fu1_w_ref[...]) + fu1_b_ref[...]),
                          fu2_w_ref[...]) + fu2_b_ref[...]
        ae_ref[...] = _mm(relu(_mm(o, ae1_w_ref[...]) + ae1_b_ref[...]),
                          ae2_w_ref[...]) + ae2_b_ref[...]


def _final(h, stats, g, b, idx3, fc1_w, fc1_b,
           st1_w, st1_b, se1_w, se1_b, fu1_w, fu1_b, ae1_w, ae1_b,
           st2_w, st2_b, se2_w, se2_b, fu2_w, fu2_b, ae2_w, ae2_b):
    full = lambda shape: pl.BlockSpec(shape, lambda i: tuple(0 for _ in shape))
    return pl.pallas_call(
        _final_body,
        grid=(GRID,),
        in_specs=[
            pl.BlockSpec((BLK, D), lambda i: (i, 0)),
            full((2, D)), full((1, D)), full((1, D)),
            pl.BlockSpec((1, 1, BLK), lambda i: (i, 0, 0)),
            full((512, D)), full((1, 512)),
            full((256, 512)), full((1, 256)), full((256, 512)), full((1, 256)),
            full((256, 512)), full((1, 256)), full((256, 512)), full((1, 256)),
            full((3, 256)), full((1, 3)), full((2, 256)), full((1, 2)),
            full((4, 256)), full((1, 4)), full((5, 256)), full((1, 5)),
        ],
        out_specs=[
            pl.BlockSpec((M, 3), lambda i: (0, 0)),
            pl.BlockSpec((M, 2), lambda i: (0, 0)),
            pl.BlockSpec((M, 4), lambda i: (0, 0)),
            pl.BlockSpec((M, 5), lambda i: (0, 0)),
        ],
        out_shape=[
            jax.ShapeDtypeStruct((M, 3), jnp.float32),
            jax.ShapeDtypeStruct((M, 2), jnp.float32),
            jax.ShapeDtypeStruct((M, 4), jnp.float32),
            jax.ShapeDtypeStruct((M, 5), jnp.float32),
        ],
        scratch_shapes=[pltpu.VMEM((M, D), jnp.float32),
                        pltpu.VMEM((M, D), jnp.float32)],
    )(h, stats, g, b, idx3, fc1_w, fc1_b,
      st1_w, st1_b, se1_w, se1_b, fu1_w, fu1_b, ae1_w, ae1_b,
      st2_w, st2_b, se2_w, se2_b, fu2_w, fu2_b, ae2_w, ae2_b)


def kernel(verts, edges, verts_idx, g0_w0, g0_b0, g0_w1, g0_b1,
           g1_w0, g1_b0, g1_w1, g1_b1, bn0_g, bn0_b, bn1_g, bn1_b,
           fc1_w, fc1_b, st1_w, st1_b, se1_w, se1_b, fu1_w, fu1_b,
           ae1_w, ae1_b, st2_w, st2_b, se2_w, se2_b, fu2_w, fu2_b,
           ae2_w, ae2_b):
    r = lambda v: v.reshape(1, -1)
    e0 = edges[:, 0]
    e1 = edges[:, 1]
    pad = P - 2 * E
    src = jnp.concatenate([e1, e0, jnp.zeros((pad,), jnp.int32)])
    dst = jnp.concatenate([e0, e1, N + (jnp.arange(pad) % (NPAD - N))])
    dst = dst.astype(jnp.int32)
    srcp = src
    dstp = dst
    zeros = jnp.zeros((NPAD, D), jnp.float32)
    idx3 = verts_idx.reshape(GRID, 1, BLK)

    out0, xn0 = _dense2(verts, g0_w0, r(g0_b0), g0_w1, r(g0_b1))
    acc = _edge_scatter(xn0, srcp, dstp, zeros)
    h1, stats1 = _combine(out0, acc)
    out1, xn1 = _bnrelu_dense2(h1, stats1, r(bn0_g), r(bn0_b),
                               g1_w0, r(g1_b0), g1_w1, r(g1_b1))
    acc2 = _edge_scatter(xn1, srcp, dstp, zeros)
    h2, stats2 = _combine(out1, acc2)
    return _final(h2, stats2, r(bn1_g), r(bn1_b), idx3, fc1_w, r(fc1_b),
                  st1_w, r(st1_b), se1_w, r(se1_b), fu1_w, r(fu1_b),
                  ae1_w, r(ae1_b), st2_w, r(st2_b), se2_w, r(se2_b),
                  fu2_w, r(fu2_b), ae2_w, r(ae2_b))


# speedup vs baseline: 2.5093x; 1.1308x over previous
import functools

import jax
import jax.numpy as jnp
from jax import lax
from jax.experimental import pallas as pl
from jax.experimental.pallas import tpu as pltpu
from jax.experimental.pallas import tpu_sc as plsc

N = 10000
E = 320000
D = 128
M = 64
EPS = 1e-5

BLK = 1000
GRID = N // BLK

NC, NS = 2, 16
NW = NC * NS
NPAD = 10112
RPT = NPAD // NS
CH = 128
CHUNKS = 157
EPT = CHUNKS * CH
P = NW * EPT
IB = 32


def _mm(x, w):
    return lax.dot_general(x, w, (((1,), (1,)), ((), ())),
                           preferred_element_type=jnp.float32)


def _scatter_body(xn_hbm, src_hbm, dst_hbm, zeros_hbm, out_hbm,
                  src_v, dst_v, rows_v, acc_sh, gi, gg, gs):
    c = lax.axis_index("c")
    s = lax.axis_index("s")
    wid = s * NC + c
    r0 = s * RPT
    pltpu.sync_copy(zeros_hbm.at[pl.ds(r0, RPT)], acc_sh.at[pl.ds(r0, RPT)])
    plsc.subcore_barrier()
    base = wid * EPT

    pltpu.sync_copy(src_hbm.at[pl.ds(base, CH)], src_v)

    def chunk(i, carry):
        boff = base + i * CH
        d = pltpu.async_copy(dst_hbm.at[pl.ds(boff, CH)], dst_v, gi)
        g = pltpu.async_copy(xn_hbm.at[src_v], rows_v, gg)
        g.wait()
        nboff = base + lax.min(i + 1, CHUNKS - 1) * CH
        pltpu.async_copy(src_hbm.at[pl.ds(nboff, CH)], src_v, gs)
        d.wait()
        pltpu.sync_copy(rows_v, acc_sh.at[dst_v], add=True)
        pltpu.make_async_copy(src_hbm.at[pl.ds(base, CH)], src_v, gs).wait()
        return carry

    lax.fori_loop(0, CHUNKS, chunk, 0)
    plsc.subcore_barrier()
    pltpu.sync_copy(acc_sh.at[pl.ds(r0, RPT)], out_hbm.at[c, pl.ds(r0, RPT)])


_edge_scatter = pl.kernel(
    _scatter_body,
    out_type=jax.ShapeDtypeStruct((NC, NPAD, D), jnp.float32),
    mesh=plsc.VectorSubcoreMesh(core_axis_name="c", subcore_axis_name="s"),
    scratch_types=[
        pltpu.VMEM((CH,), jnp.int32),
        pltpu.VMEM((CH,), jnp.int32),
        pltpu.VMEM((CH, D), jnp.float32),
        pltpu.VMEM_SHARED((NPAD, D), jnp.float32),
        pltpu.SemaphoreType.DMA,
        pltpu.SemaphoreType.DMA,
        pltpu.SemaphoreType.DMA,
    ],
)


def _dense2_body(x_ref, w0_ref, b0_ref, w1_ref, b1_ref, out0_ref, xn_ref):
    x = x_ref[...]
    out0_ref[...] = _mm(x, w0_ref[...]) + b0_ref[...]
    xn_ref[...] = _mm(x, w1_ref[...]) + b1_ref[...]


def _dense2(x, w0, b0, w1, b1):
    full = lambda shape: pl.BlockSpec(shape, lambda i: (0, 0))
    return pl.pallas_call(
        _dense2_body,
        grid=(GRID,),
        in_specs=[
            pl.BlockSpec((BLK, D), lambda i: (i, 0)),
            full((D, D)), full((1, D)), full((D, D)), full((1, D)),
        ],
        out_specs=[
            pl.BlockSpec((BLK, D), lambda i: (i, 0)),
            pl.BlockSpec((BLK, D), lambda i: (i, 0)),
        ],
        out_shape=[
            jax.ShapeDtypeStruct((N, D), jnp.float32),
            jax.ShapeDtypeStruct((N, D), jnp.float32),
        ],
    )(x, w0, b0, w1, b1)


def _combine_body(out0_ref, acc0_ref, acc1_ref, h_ref, stats_ref, sacc):
    i = pl.program_id(0)
    h = out0_ref[...] + acc0_ref[0] + acc1_ref[0]
    h_ref[...] = h

    @pl.when(i == 0)
    def _():
        sacc[...] = jnp.zeros_like(sacc)

    s1 = jnp.sum(h, axis=0, keepdims=True)
    s2 = jnp.sum(h * h, axis=0, keepdims=True)
    sacc[...] += jnp.concatenate([s1, s2], axis=0)

    @pl.when(i == GRID - 1)
    def _():
        stats_ref[...] = sacc[...]


def _combine(out0, acc):
    return pl.pallas_call(
        _combine_body,
        grid=(GRID,),
        in_specs=[
            pl.BlockSpec((BLK, D), lambda i: (i, 0)),
            pl.BlockSpec((1, BLK, D), lambda i: (0, i, 0)),
            pl.BlockSpec((1, BLK, D), lambda i: (1, i, 0)),
        ],
        out_specs=[
            pl.BlockSpec((BLK, D), lambda i: (i, 0)),
            pl.BlockSpec((2, D), lambda i: (0, 0)),
        ],
        out_shape=[
            jax.ShapeDtypeStruct((N, D), jnp.float32),
            jax.ShapeDtypeStruct((2, D), jnp.float32),
        ],
        scratch_shapes=[pltpu.VMEM((2, D), jnp.float32)],
    )(out0, acc, acc)


def _bn_apply(h, stats, g, b):
    mu = stats[0:1] / N
    var = stats[1:2] / N - mu * mu
    inv = lax.rsqrt(var + EPS)
    return jnp.maximum(g * (h - mu) * inv + b, 0.0)


def _bnrelu_dense2_body(h_ref, stats_ref, g_ref, b_ref,
                        w0_ref, b0_ref, w1_ref, b1_ref, out0_ref, xn_ref):
    x = _bn_apply(h_ref[...], stats_ref[...], g_ref[...], b_ref[...])
    out0_ref[...] = _mm(x, w0_ref[...]) + b0_ref[...]
    xn_ref[...] = _mm(x, w1_ref[...]) + b1_ref[...]


def _bnrelu_dense2(h, stats, g, b, w0, b0, w1, b1):
    full = lambda shape: pl.BlockSpec(shape, lambda i: (0, 0))
    return pl.pallas_call(
        _bnrelu_dense2_body,
        grid=(GRID,),
        in_specs=[
            pl.BlockSpec((BLK, D), lambda i: (i, 0)),
            full((2, D)), full((1, D)), full((1, D)),
            full((D, D)), full((1, D)), full((D, D)), full((1, D)),
        ],
        out_specs=[
            pl.BlockSpec((BLK, D), lambda i: (i, 0)),
            pl.BlockSpec((BLK, D), lambda i: (i, 0)),
        ],
        out_shape=[
            jax.ShapeDtypeStruct((N, D), jnp.float32),
            jax.ShapeDtypeStruct((N, D), jnp.float32),
        ],
    )(h, stats, g, b, w0, b0, w1, b1)


def _final_body(h_ref, stats_ref, g_ref, b_ref, idx_ref,
                fc1_w_ref, fc1_b_ref,
                st1_w_ref, st1_b_ref, se1_w_ref, se1_b_ref,
                fu1_w_ref, fu1_b_ref, ae1_w_ref, ae1_b_ref,
                st2_w_ref, st2_b_ref, se2_w_ref, se2_b_ref,
                fu2_w_ref, fu2_b_ref, ae2_w_ref, ae2_b_ref,
                st_ref, se_ref, fu_ref, ae_ref, seg_acc, cnt_acc):
    i = pl.program_id(0)
    x = _bn_apply(h_ref[...], stats_ref[...], g_ref[...], b_ref[...])

    @pl.when(i == 0)
    def _():
        seg_acc[...] = jnp.zeros_like(seg_acc)
        cnt_acc[...] = jnp.zeros_like(cnt_acc)

    idx = idx_ref[0]
    seg_ids = lax.broadcasted_iota(jnp.int32, (M, BLK), 0)
    onehot = (seg_ids == idx).astype(jnp.float32)
    seg_acc[...] += jnp.dot(onehot, x, preferred_element_type=jnp.float32)
    cnt = jnp.sum(onehot, axis=1, keepdims=True)
    cnt_acc[...] += jnp.broadcast_to(cnt, (M, D))

    @pl.when(i == GRID - 1)
    def _():
        mean = seg_acc[...] / jnp.maximum(cnt_acc[...], 1.0)
        relu = lambda v: jnp.maximum(v, 0.0)
        o = relu(_mm(mean, fc1_w_ref[...]) + fc1_b_ref[...])
        st_ref[...] = _mm(relu(_mm(o, st1_w_ref[...]) + st1_b_ref[...]),
                          st2_w_ref[...]) + st2_b_ref[...]
        se_ref[...] = _mm(relu(_mm(o, se1_w_ref[...]) + se1_b_ref[...]),
                          se2_w_ref[...]) + se2_b_ref[...]
        fu_ref[...] = _mm(relu(_mm(o, fu1_w_ref[...]) + fu1_b_ref[...]),
                          fu2_w_ref[...]) + fu2_b_ref[...]
        ae_ref[...] = _mm(relu(_mm(o, ae1_w_ref[...]) + ae1_b_ref[...]),
                          ae2_w_ref[...]) + ae2_b_ref[...]


def _final(h, stats, g, b, idx3, fc1_w, fc1_b,
           st1_w, st1_b, se1_w, se1_b, fu1_w, fu1_b, ae1_w, ae1_b,
           st2_w, st2_b, se2_w, se2_b, fu2_w, fu2_b, ae2_w, ae2_b):
    full = lambda shape: pl.BlockSpec(shape, lambda i: tuple(0 for _ in shape))
    return pl.pallas_call(
        _final_body,
        grid=(GRID,),
        in_specs=[
            pl.BlockSpec((BLK, D), lambda i: (i, 0)),
            full((2, D)), full((1, D)), full((1, D)),
            pl.BlockSpec((1, 1, BLK), lambda i: (i, 0, 0)),
            full((512, D)), full((1, 512)),
            full((256, 512)), full((1, 256)), full((256, 512)), full((1, 256)),
            full((256, 512)), full((1, 256)), full((256, 512)), full((1, 256)),
            full((3, 256)), full((1, 3)), full((2, 256)), full((1, 2)),
            full((4, 256)), full((1, 4)), full((5, 256)), full((1, 5)),
        ],
        out_specs=[
            pl.BlockSpec((M, 3), lambda i: (0, 0)),
            pl.BlockSpec((M, 2), lambda i: (0, 0)),
            pl.BlockSpec((M, 4), lambda i: (0, 0)),
            pl.BlockSpec((M, 5), lambda i: (0, 0)),
        ],
        out_shape=[
            jax.ShapeDtypeStruct((M, 3), jnp.float32),
            jax.ShapeDtypeStruct((M, 2), jnp.float32),
            jax.ShapeDtypeStruct((M, 4), jnp.float32),
            jax.ShapeDtypeStruct((M, 5), jnp.float32),
        ],
        scratch_shapes=[pltpu.VMEM((M, D), jnp.float32),
                        pltpu.VMEM((M, D), jnp.float32)],
    )(h, stats, g, b, idx3, fc1_w, fc1_b,
      st1_w, st1_b, se1_w, se1_b, fu1_w, fu1_b, ae1_w, ae1_b,
      st2_w, st2_b, se2_w, se2_b, fu2_w, fu2_b, ae2_w, ae2_b)


def kernel(verts, edges, verts_idx, g0_w0, g0_b0, g0_w1, g0_b1,
           g1_w0, g1_b0, g1_w1, g1_b1, bn0_g, bn0_b, bn1_g, bn1_b,
           fc1_w, fc1_b, st1_w, st1_b, se1_w, se1_b, fu1_w, fu1_b,
           ae1_w, ae1_b, st2_w, st2_b, se2_w, se2_b, fu2_w, fu2_b,
           ae2_w, ae2_b):
    r = lambda v: v.reshape(1, -1)
    e0 = edges[:, 0]
    e1 = edges[:, 1]
    pad = P - 2 * E
    src = jnp.concatenate([e1, e0, jnp.zeros((pad,), jnp.int32)])
    dst = jnp.concatenate([e0, e1, N + (jnp.arange(pad) % (NPAD - N))])
    dst = dst.astype(jnp.int32)
    srcp = src
    dstp = dst
    zeros = jnp.zeros((NPAD, D), jnp.float32)
    idx3 = verts_idx.reshape(GRID, 1, BLK)

    out0, xn0 = _dense2(verts, g0_w0, r(g0_b0), g0_w1, r(g0_b1))
    acc = _edge_scatter(xn0, srcp, dstp, zeros)
    h1, stats1 = _combine(out0, acc)
    out1, xn1 = _bnrelu_dense2(h1, stats1, r(bn0_g), r(bn0_b),
                               g1_w0, r(g1_b0), g1_w1, r(g1_b1))
    acc2 = _edge_scatter(xn1, srcp, dstp, zeros)
    h2, stats2 = _combine(out1, acc2)
    return _final(h2, stats2, r(bn1_g), r(bn1_b), idx3, fc1_w, r(fc1_b),
                  st1_w, r(st1_b), se1_w, r(se1_b), fu1_w, r(fu1_b),
                  ae1_w, r(ae1_b), st2_w, r(st2_b), se2_w, r(se2_b),
                  fu2_w, r(fu2_b), ae2_w, r(ae2_b))
